# Initial kernel scaffold; baseline (speedup 1.0000x reference)
import jax
import jax.numpy as jnp
from jax.experimental import pallas as pl


def kernel(x, xyz, W, b, gamma, beta):
    raise NotImplementedError("write your pallas kernel here")


# speedup vs baseline: 12.3571x; 12.3571x over previous
import functools

import jax
import jax.numpy as jnp
from jax import lax
from jax.experimental import pallas as pl
from jax.experimental.pallas import tpu as pltpu
from jax.experimental.pallas import tpu_sc as plsc

K_NB = 16


def _topk_body(n_total, rblk, xyz_ref, xyzt_ref, idx_ref):
    b = pl.program_id(0)
    xr = xyz_ref[0]
    xt = xyzt_ref[0]
    xx_r = (xr[:, 0:1] * xr[:, 0:1] + xr[:, 1:2] * xr[:, 1:2]) + xr[:, 2:3] * xr[:, 2:3]
    xx_c = (xt[0:1, :] * xt[0:1, :] + xt[1:2, :] * xt[1:2, :]) + xt[2:3, :] * xt[2:3, :]
    dot = lax.dot_general(xr, xt, (((1,), (0,)), ((), ())),
                          preferred_element_type=jnp.float32)
    d = ((-xx_r) + 2.0 * dot) - xx_c
    iota = lax.broadcasted_iota(jnp.int32, (rblk, n_total), 1)
    cols = []
    for t in range(K_NB):
        m = jnp.max(d, axis=1, keepdims=True)
        am = jnp.min(jnp.where(d == m, iota, n_total), axis=1, keepdims=True)
        cols.append(am)
        if t < K_NB - 1:
            d = jnp.where(iota == am, -jnp.inf, d)
    idx_ref[0] = jnp.concatenate(cols, axis=1) + b * n_total


def _topk_indices(xyz):
    B, N, _ = xyz.shape
    R = 128
    xyzt = jnp.transpose(xyz, (0, 2, 1))
    return pl.pallas_call(
        functools.partial(_topk_body, N, R),
        grid=(B, N // R),
        in_specs=[
            pl.BlockSpec((1, R, 3), lambda b, i: (b, i, 0)),
            pl.BlockSpec((1, 3, N), lambda b, i: (b, 0, 0)),
        ],
        out_specs=pl.BlockSpec((1, R, K_NB), lambda b, i: (b, i, 0)),
        out_shape=jax.ShapeDtypeStruct((B, N, K_NB), jnp.int32),
    )(xyz, xyzt)


def _proj_body(c_in, x_ref, w_ref, a_ref, bf_ref):
    w1 = w_ref[:, :c_in]
    w2 = w_ref[:, c_in:]
    xb = x_ref[...]
    dn = (((1,), (1,)), ((), ()))
    a_ref[...] = lax.dot_general(xb, w1 - w2, dn, preferred_element_type=jnp.float32)
    bf_ref[...] = lax.dot_general(xb, w2, dn, preferred_element_type=jnp.float32)


def _projections(x2d, W):
    BN, C = x2d.shape
    Cout = W.shape[0]
    R = 512
    return pl.pallas_call(
        functools.partial(_proj_body, C),
        grid=(BN // R,),
        in_specs=[
            pl.BlockSpec((R, C), lambda i: (i, 0)),
            pl.BlockSpec(W.shape, lambda i: (0, 0)),
        ],
        out_specs=[
            pl.BlockSpec((R, Cout), lambda i: (i, 0)),
            pl.BlockSpec((R, Cout), lambda i: (i, 0)),
        ],
        out_shape=[
            jax.ShapeDtypeStruct((BN, Cout), jnp.float32),
            jax.ShapeDtypeStruct((BN, Cout), jnp.float32),
        ],
    )(x2d, W)


def _gather_reduce(bf, idx_flat):
    BN, C = bf.shape
    info = plsc.get_sparse_core_info()
    NC, NS = info.num_cores, info.num_subcores
    NW = NC * NS
    pts_per_w = BN // NW
    CP = 8
    n_chunks = pts_per_w // CP
    ngrp = C // 16
    mesh = plsc.VectorSubcoreMesh(core_axis_name="c", subcore_axis_name="s")

    @functools.partial(
        pl.kernel,
        mesh=mesh,
        out_type=[
            jax.ShapeDtypeStruct((BN, C), jnp.float32),
            jax.ShapeDtypeStruct((BN, C), jnp.float32),
            jax.ShapeDtypeStruct((BN, C), jnp.float32),
            jax.ShapeDtypeStruct((NW, C), jnp.float32),
        ],
        scratch_types=[
            pltpu.VMEM((CP * K_NB,), jnp.int32),
            pltpu.VMEM((CP * K_NB, C), jnp.float32),
            pltpu.VMEM((CP, C), jnp.float32),
            pltpu.VMEM((CP, C), jnp.float32),
            pltpu.VMEM((CP, C), jnp.float32),
            pltpu.VMEM((C,), jnp.float32),
            pltpu.SemaphoreType.DMA,
        ],
    )
    def _sc(bf_hbm, idx_hbm, s_hbm, mx_hbm, mn_hbm, q_hbm,
            idx_v, rows_v, s_v, mx_v, mn_v, q_acc, sem):
        wid = lax.axis_index("s") * NC + lax.axis_index("c")
        for g in range(ngrp):
            q_acc[pl.ds(g * 16, 16)] = jnp.zeros((16,), jnp.float32)

        def chunk_body(c, carry):
            p0 = wid * pts_per_w + c * CP
            pltpu.sync_copy(idx_hbm.at[pl.ds(p0 * K_NB, CP * K_NB)], idx_v)
            pltpu.async_copy(bf_hbm.at[idx_v], rows_v, sem).wait()

            def point_body(p, carry2):
                r0 = p * K_NB
                for g in range(ngrp):
                    sl = pl.ds(g * 16, 16)
                    v = rows_v[r0, sl]
                    s = v
                    mx = v
                    mn = v
                    q = v * v
                    for j in range(1, K_NB):
                        v = rows_v[r0 + j, sl]
                        s = s + v
                        mx = jnp.maximum(mx, v)
                        mn = jnp.minimum(mn, v)
                        q = q + v * v
                    s_v[p, sl] = s
                    mx_v[p, sl] = mx
                    mn_v[p, sl] = mn
                    q_acc[sl] = q_acc[sl] + q
                return carry2

            lax.fori_loop(0, CP, point_body, 0)
            pltpu.sync_copy(s_v, s_hbm.at[pl.ds(p0, CP)])
            pltpu.sync_copy(mx_v, mx_hbm.at[pl.ds(p0, CP)])
            pltpu.sync_copy(mn_v, mn_hbm.at[pl.ds(p0, CP)])
            return carry

        lax.fori_loop(0, n_chunks, chunk_body, 0)
        pltpu.sync_copy(q_acc, q_hbm.at[wid])

    return _sc(bf, idx_flat)


def _stats_body(nblk, a_ref, s_ref, out_ref, acc_ref):
    i = pl.program_id(0)

    @pl.when(i == 0)
    def _():
        acc_ref[...] = jnp.zeros_like(acc_ref)

    a = a_ref[...]
    s = s_ref[...]
    acc_ref[0:1, :] = acc_ref[0:1, :] + jnp.sum(a, axis=0, keepdims=True)
    acc_ref[1:2, :] = acc_ref[1:2, :] + jnp.sum(a * a, axis=0, keepdims=True)
    acc_ref[2:3, :] = acc_ref[2:3, :] + jnp.sum(a * s, axis=0, keepdims=True)
    acc_ref[3:4, :] = acc_ref[3:4, :] + jnp.sum(s, axis=0, keepdims=True)

    @pl.when(i == nblk - 1)
    def _():
        out_ref[...] = acc_ref[...]


def _channel_stats(a, s):
    BN, C = a.shape
    R = 1024
    nblk = BN // R
    return pl.pallas_call(
        functools.partial(_stats_body, nblk),
        grid=(nblk,),
        in_specs=[
            pl.BlockSpec((R, C), lambda i: (i, 0)),
            pl.BlockSpec((R, C), lambda i: (i, 0)),
        ],
        out_specs=pl.BlockSpec((8, C), lambda i: (0, 0)),
        out_shape=jax.ShapeDtypeStruct((8, C), jnp.float32),
        scratch_shapes=[pltpu.VMEM((8, C), jnp.float32)],
    )(a, s)


def _apply_body(n_edges, a_ref, mx_ref, mn_ref, st_ref, qp_ref, g_ref, be_ref, o_ref):
    st = st_ref[...]
    sum_a = st[0:1, :]
    sum_a2 = st[1:2, :]
    sum_as = st[2:3, :]
    sum_s = st[3:4, :]
    sum_q = jnp.sum(qp_ref[...], axis=0, keepdims=True)
    e = float(n_edges)
    mean = (K_NB * sum_a + sum_s) / e
    ex2 = (K_NB * sum_a2 + 2.0 * sum_as + sum_q) / e
    var = ex2 - mean * mean
    rstd = 1.0 / jnp.sqrt(var + 1e-5)
    g = g_ref[...]
    be = be_ref[...]
    m = jnp.where(g >= 0.0, mx_ref[...], mn_ref[...])
    z = (a_ref[...] + m - mean) * rstd
    o_ref[...] = jnp.maximum(g * z + be, 0.0)


def _apply(a, mx, mn, stats, qp, gamma, beta, n_edges):
    BN, C = a.shape
    R = 1024
    blk = pl.BlockSpec((R, C), lambda i: (i, 0))
    return pl.pallas_call(
        functools.partial(_apply_body, n_edges),
        grid=(BN // R,),
        in_specs=[
            blk, blk, blk,
            pl.BlockSpec((8, C), lambda i: (0, 0)),
            pl.BlockSpec(qp.shape, lambda i: (0, 0)),
            pl.BlockSpec((1, C), lambda i: (0, 0)),
            pl.BlockSpec((1, C), lambda i: (0, 0)),
        ],
        out_specs=blk,
        out_shape=jax.ShapeDtypeStruct((BN, C), jnp.float32),
    )(a, mx, mn, stats, qp, gamma, beta)


def kernel(x, xyz, W, b, gamma, beta):
    B, N, C = x.shape
    Cout = W.shape[0]
    del b

    idx = _topk_indices(xyz)
    a2d, bf2d = _projections(x.reshape(B * N, C), W)
    s, mx, mn, qp = _gather_reduce(bf2d, idx.reshape(-1))
    stats = _channel_stats(a2d, s)
    out = _apply(a2d, mx, mn, stats, qp,
                 gamma.reshape(1, Cout), beta.reshape(1, Cout), B * N * K_NB)
    return out.reshape(B, N, Cout)
